# baseline (device time: 9571 ns/iter reference)
import jax
import jax.numpy as jnp
from jax import lax
from jax.experimental import pallas as pl
from jax.experimental.pallas import tpu as pltpu

M = 256
N = 256
CHUNKS = 8
R = M // CHUNKS


def kernel(x):
    def body(x_ref, out_ref, xbf_ref, comm_x_ref,
             send_x, recv_x, send_y, recv_y):
        my_x = lax.axis_index("x")
        my_y = lax.axis_index("y")
        x_nbr = (1 - my_x, my_y)
        y_nbr = (my_x, 1 - my_y)

        barrier_sem = pltpu.get_barrier_semaphore()
        for nbr in (x_nbr, y_nbr):
            pl.semaphore_signal(barrier_sem, inc=1, device_id=nbr,
                                device_id_type=pl.DeviceIdType.MESH)
        xbf_ref[:, :] = x_ref[:, :].astype(jnp.bfloat16)
        pl.semaphore_wait(barrier_sem, 2)

        my_col = out_ref.at[:, pl.ds(my_y * N, N)]

        rdx = []
        for c in range(CHUNKS):
            rows = pl.ds(c * R, R)
            r = pltpu.make_async_remote_copy(
                src_ref=xbf_ref.at[rows, :], dst_ref=comm_x_ref.at[rows, :],
                send_sem=send_x.at[c], recv_sem=recv_x.at[c],
                device_id=x_nbr, device_id_type=pl.DeviceIdType.MESH)
            r.start()
            rdx.append(r)

        rdy = []
        for c in range(CHUNKS):
            rows = pl.ds(c * R, R)
            rdx[c].wait_recv()
            my_col[rows, :] = xbf_ref[rows, :] + comm_x_ref[rows, :]
            r = pltpu.make_async_remote_copy(
                src_ref=my_col.at[rows, :], dst_ref=my_col.at[rows, :],
                send_sem=send_y.at[c], recv_sem=recv_y.at[c],
                device_id=y_nbr, device_id_type=pl.DeviceIdType.MESH)
            r.start()
            rdy.append(r)

        for c in range(CHUNKS):
            rdy[c].wait_recv()
        for c in range(CHUNKS):
            rdx[c].wait_send()
            rdy[c].wait_send()

    return pl.pallas_call(
        body,
        out_shape=jax.ShapeDtypeStruct((M, 2 * N), jnp.bfloat16),
        in_specs=[pl.BlockSpec(memory_space=pltpu.VMEM)],
        out_specs=pl.BlockSpec(memory_space=pltpu.VMEM),
        scratch_shapes=[
            pltpu.VMEM((M, N), jnp.bfloat16),
            pltpu.VMEM((M, N), jnp.bfloat16),
            pltpu.SemaphoreType.DMA((CHUNKS,)),
            pltpu.SemaphoreType.DMA((CHUNKS,)),
            pltpu.SemaphoreType.DMA((CHUNKS,)),
            pltpu.SemaphoreType.DMA((CHUNKS,)),
        ],
        compiler_params=pltpu.CompilerParams(collective_id=0),
    )(x)
